# baseline (speedup 1.0000x reference)
import functools

import jax
import jax.numpy as jnp
from jax import lax
from jax.experimental import pallas as pl
from jax.experimental.pallas import tpu as pltpu
from jax.experimental.pallas import tpu_sc as plsc

N = 1_000_000
NW = 32
L = 16
BASE = 31_248
CHUNK = 1_024
FULL_CHUNKS = 30
TAIL = BASE - FULL_CHUNKS * CHUNK
EXTRA_START = NW * BASE
EXTRA = N - EXTRA_START


def _rsqrt_nr(a):
    i = lax.bitcast_convert_type(a, jnp.int32)
    i = jnp.int32(0x5F3759DF) - (i >> 1)
    y = lax.bitcast_convert_type(i, jnp.float32)
    ah = 0.5 * a
    for _ in range(3):
        y = y * (1.5 - ah * y * y)
    return y


def _compute_group(b, pos_v, ls_v, rot_v, dr_v, di_v, out_v):
    lane = lax.iota(jnp.int32, L)
    row3 = 3 * b + 3 * lane
    row4 = 4 * b + 4 * lane

    px = plsc.load_gather(pos_v, [row3])
    py = plsc.load_gather(pos_v, [row3 + 1])
    pz = plsc.load_gather(pos_v, [row3 + 2])
    sx = jnp.exp(plsc.load_gather(ls_v, [row3]))
    sy = jnp.exp(plsc.load_gather(ls_v, [row3 + 1]))
    sz = jnp.exp(plsc.load_gather(ls_v, [row3 + 2]))
    qw = plsc.load_gather(rot_v, [row4])
    qx = plsc.load_gather(rot_v, [row4 + 1])
    qy = plsc.load_gather(rot_v, [row4 + 2])
    qz = plsc.load_gather(rot_v, [row4 + 3])
    dr = dr_v[pl.ds(b, L)]
    di = di_v[pl.ds(b, L)]

    n2 = qw * qw + qx * qx + qy * qy + qz * qz
    norm = n2 * _rsqrt_nr(n2)
    inv = 1.0 / (norm + 1e-8)
    w, x, y, z = qw * inv, qx * inv, qy * inv, qz * inv

    xx, yy, zz = x * x, y * y, z * z
    xy, xz, yz = x * y, x * z, y * z
    wx, wy, wz = w * x, w * y, w * z
    r00 = 1.0 - 2.0 * (yy + zz)
    r01 = 2.0 * (xy - wz)
    r02 = 2.0 * (xz + wy)
    r10 = 2.0 * (xy + wz)
    r11 = 1.0 - 2.0 * (xx + zz)
    r12 = 2.0 * (yz - wx)
    r20 = 2.0 * (xz - wy)
    r21 = 2.0 * (yz + wx)
    r22 = 1.0 - 2.0 * (xx + yy)

    s2x, s2y, s2z = sx * sx, sy * sy, sz * sz
    a00, a01, a02 = r00 * s2x, r01 * s2y, r02 * s2z
    a10, a11, a12 = r10 * s2x, r11 * s2y, r12 * s2z
    a20, a21, a22 = r20 * s2x, r21 * s2y, r22 * s2z
    s00 = a00 * r00 + a01 * r01 + a02 * r02
    s01 = a00 * r10 + a01 * r11 + a02 * r12
    s02 = a00 * r20 + a01 * r21 + a02 * r22
    s11 = a10 * r10 + a11 * r11 + a12 * r12
    s12 = a10 * r20 + a11 * r21 + a12 * r22
    s22 = a20 * r20 + a21 * r21 + a22 * r22

    t = dr * dr + di * di + 1e-12
    dmag = t * _rsqrt_nr(t)

    row16 = 16 * b + 16 * lane
    vals = (px, py, pz,
            s00, s01, s02, s01, s11, s12, s02, s12, s22,
            sx, sy, sz, dmag)
    for c in range(16):
        plsc.store_scatter(out_v, [row16 + c], vals[c])


def _process_chunk(start_c, npts,
                   pos_hbm, ls_hbm, rot_hbm, dr_hbm, di_hbm, out_hbm,
                   pos_v, ls_v, rot_v, dr_v, di_v, out_v):
    pltpu.sync_copy(pos_hbm.at[pl.ds(start_c * 3, npts * 3)],
                    pos_v.at[pl.ds(0, npts * 3)])
    pltpu.sync_copy(ls_hbm.at[pl.ds(start_c * 3, npts * 3)],
                    ls_v.at[pl.ds(0, npts * 3)])
    pltpu.sync_copy(rot_hbm.at[pl.ds(start_c * 4, npts * 4)],
                    rot_v.at[pl.ds(0, npts * 4)])
    pltpu.sync_copy(dr_hbm.at[pl.ds(start_c, npts)], dr_v.at[pl.ds(0, npts)])
    pltpu.sync_copy(di_hbm.at[pl.ds(start_c, npts)], di_v.at[pl.ds(0, npts)])

    def body(g, carry):
        _compute_group(g * L, pos_v, ls_v, rot_v, dr_v, di_v, out_v)
        return carry
    lax.fori_loop(0, npts // L, body, 0)

    pltpu.sync_copy(out_v.at[pl.ds(0, npts * 16)],
                    out_hbm.at[pl.ds(start_c * 16, npts * 16)])


def _sc_kernel(pos_hbm, ls_hbm, rot_hbm, dr_hbm, di_hbm, out_hbm,
               pos_v, ls_v, rot_v, dr_v, di_v, out_v):
    wid = lax.axis_index("s") * 2 + lax.axis_index("c")
    start = wid * BASE

    chunk = functools.partial(
        _process_chunk,
        pos_hbm=pos_hbm, ls_hbm=ls_hbm, rot_hbm=rot_hbm,
        dr_hbm=dr_hbm, di_hbm=di_hbm, out_hbm=out_hbm,
        pos_v=pos_v, ls_v=ls_v, rot_v=rot_v,
        dr_v=dr_v, di_v=di_v, out_v=out_v)

    def body(g, carry):
        chunk(start + g * CHUNK, CHUNK)
        return carry
    lax.fori_loop(0, FULL_CHUNKS, body, 0)
    chunk(start + FULL_CHUNKS * CHUNK, TAIL)

    @pl.when(wid == NW - 1)
    def _():
        chunk(EXTRA_START, EXTRA)


def kernel(positions, log_scales, rotations, density_real, density_imag):
    mesh = plsc.VectorSubcoreMesh(core_axis_name="c", subcore_axis_name="s")
    f = pl.kernel(
        _sc_kernel,
        out_type=jax.ShapeDtypeStruct((N * 16,), jnp.float32),
        mesh=mesh,
        compiler_params=pltpu.CompilerParams(needs_layout_passes=False),
        scratch_types=[
            pltpu.VMEM((CHUNK * 3,), jnp.float32),
            pltpu.VMEM((CHUNK * 3,), jnp.float32),
            pltpu.VMEM((CHUNK * 4,), jnp.float32),
            pltpu.VMEM((CHUNK,), jnp.float32),
            pltpu.VMEM((CHUNK,), jnp.float32),
            pltpu.VMEM((CHUNK * 16,), jnp.float32),
        ],
    )
    out = f(positions.reshape(-1), log_scales.reshape(-1),
            rotations.reshape(-1), density_real, density_imag)
    return out.reshape(N, 16)


# speedup vs baseline: 1.0150x; 1.0150x over previous
import jax
import jax.numpy as jnp
from jax import lax
from jax.experimental import pallas as pl
from jax.experimental.pallas import tpu as pltpu
from jax.experimental.pallas import tpu_sc as plsc

N = 1_000_000
NW = 32
L = 16
CHUNK = 1_024
NCHUNKS = 977
LAST_START = N - CHUNK
SLOTS = 31


def _rsqrt_nr(a):
    i = lax.bitcast_convert_type(a, jnp.int32)
    i = jnp.int32(0x5F3759DF) - (i >> 1)
    y = lax.bitcast_convert_type(i, jnp.float32)
    ah = 0.5 * a
    for _ in range(3):
        y = y * (1.5 - ah * y * y)
    return y


def _compute_group(b, pos_v, ls_v, rot_v, dr_v, di_v, out_v):
    lane = lax.iota(jnp.int32, L)
    row3 = 3 * b + 3 * lane
    row4 = 4 * b + 4 * lane

    px = plsc.load_gather(pos_v, [row3])
    py = plsc.load_gather(pos_v, [row3 + 1])
    pz = plsc.load_gather(pos_v, [row3 + 2])
    sx = jnp.exp(plsc.load_gather(ls_v, [row3]))
    sy = jnp.exp(plsc.load_gather(ls_v, [row3 + 1]))
    sz = jnp.exp(plsc.load_gather(ls_v, [row3 + 2]))
    qw = plsc.load_gather(rot_v, [row4])
    qx = plsc.load_gather(rot_v, [row4 + 1])
    qy = plsc.load_gather(rot_v, [row4 + 2])
    qz = plsc.load_gather(rot_v, [row4 + 3])
    dr = dr_v[pl.ds(b, L)]
    di = di_v[pl.ds(b, L)]

    n2 = qw * qw + qx * qx + qy * qy + qz * qz
    norm = n2 * _rsqrt_nr(n2)
    inv = 1.0 / (norm + 1e-8)
    w, x, y, z = qw * inv, qx * inv, qy * inv, qz * inv

    xx, yy, zz = x * x, y * y, z * z
    xy, xz, yz = x * y, x * z, y * z
    wx, wy, wz = w * x, w * y, w * z
    r00 = 1.0 - 2.0 * (yy + zz)
    r01 = 2.0 * (xy - wz)
    r02 = 2.0 * (xz + wy)
    r10 = 2.0 * (xy + wz)
    r11 = 1.0 - 2.0 * (xx + zz)
    r12 = 2.0 * (yz - wx)
    r20 = 2.0 * (xz - wy)
    r21 = 2.0 * (yz + wx)
    r22 = 1.0 - 2.0 * (xx + yy)

    s2x, s2y, s2z = sx * sx, sy * sy, sz * sz
    a00, a01, a02 = r00 * s2x, r01 * s2y, r02 * s2z
    a10, a11, a12 = r10 * s2x, r11 * s2y, r12 * s2z
    a20, a21, a22 = r20 * s2x, r21 * s2y, r22 * s2z
    s00 = a00 * r00 + a01 * r01 + a02 * r02
    s01 = a00 * r10 + a01 * r11 + a02 * r12
    s02 = a00 * r20 + a01 * r21 + a02 * r22
    s11 = a10 * r10 + a11 * r11 + a12 * r12
    s12 = a10 * r20 + a11 * r21 + a12 * r22
    s22 = a20 * r20 + a21 * r21 + a22 * r22

    t = dr * dr + di * di + 1e-12
    dmag = t * _rsqrt_nr(t)

    row16 = 16 * b + 16 * lane
    vals = (px, py, pz,
            s00, s01, s02, s01, s11, s12, s02, s12, s22,
            sx, sy, sz, dmag)
    for c in range(16):
        plsc.store_scatter(out_v, [row16 + c], vals[c])


def _sc_kernel(pos_hbm, ls_hbm, rot_hbm, dr_hbm, di_hbm, out_hbm,
               pos_v, ls_v, rot_v, dr_v, di_v, out_v,
               sem_in, sem_out):
    wid = lax.axis_index("s") * 2 + lax.axis_index("c")

    def chunk_start(slot):
        return jnp.minimum((wid + NW * slot) * CHUNK, LAST_START)

    def in_descs(start, b):
        return [
            pltpu.make_async_copy(pos_hbm.at[pl.ds(start * 3, CHUNK * 3)],
                                  pos_v[b], sem_in[b]),
            pltpu.make_async_copy(ls_hbm.at[pl.ds(start * 3, CHUNK * 3)],
                                  ls_v[b], sem_in[b]),
            pltpu.make_async_copy(rot_hbm.at[pl.ds(start * 4, CHUNK * 4)],
                                  rot_v[b], sem_in[b]),
            pltpu.make_async_copy(dr_hbm.at[pl.ds(start, CHUNK)],
                                  dr_v[b], sem_in[b]),
            pltpu.make_async_copy(di_hbm.at[pl.ds(start, CHUNK)],
                                  di_v[b], sem_in[b]),
        ]

    def out_desc(start, b):
        return pltpu.make_async_copy(
            out_v[b], out_hbm.at[pl.ds(start * 16, CHUNK * 16)], sem_out[b])

    def issue_in(slot, b):
        for d in in_descs(chunk_start(slot), b):
            d.start()

    def wait_in(slot, b):
        for d in in_descs(chunk_start(slot), b):
            d.wait()

    issue_in(0, 0)

    def body(t, carry):
        for b in (0, 1):
            g = 2 * t + b

            @pl.when(g < SLOTS)
            def _():
                wait_in(g, b)

                @pl.when(g + 1 < SLOTS)
                def _():
                    issue_in(g + 1, 1 - b)

                @pl.when(g >= 2)
                def _():
                    out_desc(chunk_start(g - 2), b).wait()

                def grp(i, c):
                    _compute_group(i * L, pos_v[b], ls_v[b], rot_v[b],
                                   dr_v[b], di_v[b], out_v[b])
                    return c
                lax.fori_loop(0, CHUNK // L, grp, 0)

                out_desc(chunk_start(g), b).start()
        return carry

    lax.fori_loop(0, (SLOTS + 1) // 2, body, 0)

    out_desc(chunk_start(SLOTS - 2), (SLOTS - 2) % 2).wait()
    out_desc(chunk_start(SLOTS - 1), (SLOTS - 1) % 2).wait()


def kernel(positions, log_scales, rotations, density_real, density_imag):
    mesh = plsc.VectorSubcoreMesh(core_axis_name="c", subcore_axis_name="s")
    f = pl.kernel(
        _sc_kernel,
        out_type=jax.ShapeDtypeStruct((N * 16,), jnp.float32),
        mesh=mesh,
        compiler_params=pltpu.CompilerParams(needs_layout_passes=False),
        scratch_types=[
            [pltpu.VMEM((CHUNK * 3,), jnp.float32) for _ in range(2)],
            [pltpu.VMEM((CHUNK * 3,), jnp.float32) for _ in range(2)],
            [pltpu.VMEM((CHUNK * 4,), jnp.float32) for _ in range(2)],
            [pltpu.VMEM((CHUNK,), jnp.float32) for _ in range(2)],
            [pltpu.VMEM((CHUNK,), jnp.float32) for _ in range(2)],
            [pltpu.VMEM((CHUNK * 16,), jnp.float32) for _ in range(2)],
            [pltpu.SemaphoreType.DMA for _ in range(2)],
            [pltpu.SemaphoreType.DMA for _ in range(2)],
        ],
    )
    out = f(positions.reshape(-1), log_scales.reshape(-1),
            rotations.reshape(-1), density_real, density_imag)
    return out.reshape(N, 16)
